# Initial kernel scaffold; baseline (speedup 1.0000x reference)
import jax
import jax.numpy as jnp
from jax.experimental import pallas as pl


def kernel(x, edge_index, W, b):
    raise NotImplementedError("write your pallas kernel here")


# speedup vs baseline: 27.1559x; 27.1559x over previous
import functools

import jax
import jax.numpy as jnp
from jax import lax
from jax.experimental import pallas as pl
from jax.experimental.pallas import tpu as pltpu
from jax.experimental.pallas import tpu_sc as plsc

N = 10000
E = 320000
F = 128
H = 128

NC = 2
NS = 16
NW = NC * NS
CHUNK = 80
EPW = E // NW
CPW = EPW // CHUNK

_MESH = plsc.VectorSubcoreMesh(core_axis_name="c", subcore_axis_name="s")
_SC_PARAMS = pltpu.CompilerParams(use_tc_tiling_on_sc=False)


@functools.partial(
    pl.kernel,
    out_type=jax.ShapeDtypeStruct((NC, N), jnp.float32),
    mesh=_MESH,
    compiler_params=_SC_PARAMS,
    scratch_types=[
        pltpu.VMEM((CPW, CHUNK), jnp.int32),
        pltpu.VMEM((CHUNK,), jnp.float32),
        pltpu.VMEM((1024,), jnp.float32),
        pltpu.VMEM_SHARED((N,), jnp.float32),
    ],
)
def _deg_kernel(dst_hbm, deg_out, idx_v, ones_v, zb_v, deg_sh):
    c = lax.axis_index("c")
    s = lax.axis_index("s")
    wid = s * NC + c

    def _z(i, _):
        zb_v[pl.ds(i * 16, 16)] = jnp.zeros((16,), jnp.float32)
        return 0
    lax.fori_loop(0, 64, _z, 0)
    for i in range(CHUNK // 16):
        ones_v[pl.ds(i * 16, 16)] = jnp.ones((16,), jnp.float32)

    @pl.when(s < 10)
    def _():
        pltpu.sync_copy(zb_v.at[pl.ds(0, 1000)], deg_sh.at[pl.ds(s * 1000, 1000)])
    plsc.subcore_barrier()

    pltpu.sync_copy(dst_hbm.at[pl.ds(wid * CPW, CPW)], idx_v)

    def _chunk(g, _):
        pltpu.sync_copy(ones_v, deg_sh.at[idx_v.at[g]], add=True)
        return 0
    lax.fori_loop(0, CPW, _chunk, 0)
    plsc.subcore_barrier()

    @pl.when(s < 10)
    def _():
        pltpu.sync_copy(deg_sh.at[pl.ds(s * 1000, 1000)], zb_v.at[pl.ds(0, 1000)])
        pltpu.sync_copy(zb_v.at[pl.ds(0, 1000)], deg_out.at[c, pl.ds(s * 1000, 1000)])


def _dense_body(x_ref, w_ref, parts_ref, y_ref):
    deg = 1.0 + parts_ref[0] + parts_ref[1]
    dis = lax.rsqrt(deg)
    xw = jnp.dot(x_ref[...], w_ref[...], preferred_element_type=jnp.float32)
    y_ref[...] = xw * dis


_dense = pl.pallas_call(
    _dense_body,
    out_shape=jax.ShapeDtypeStruct((N, H), jnp.float32),
)


@functools.partial(
    pl.kernel,
    out_type=jax.ShapeDtypeStruct((NC, N, H), jnp.float32),
    mesh=_MESH,
    compiler_params=_SC_PARAMS,
    scratch_types=[
        pltpu.VMEM((CPW, CHUNK), jnp.int32),
        pltpu.VMEM((CPW, CHUNK), jnp.int32),
        pltpu.VMEM((CHUNK, H), jnp.float32),
        pltpu.VMEM((125, H), jnp.float32),
        pltpu.VMEM_SHARED((N, H), jnp.float32),
    ],
)
def _msg_kernel(y_hbm, src_hbm, dst_hbm, acc_out, sidx, didx, rows, zb, acc_sh):
    c = lax.axis_index("c")
    s = lax.axis_index("s")
    wid = s * NC + c

    def _z(t, _):
        i = t // 8
        j = t % 8
        zb[i, pl.ds(j * 16, 16)] = jnp.zeros((16,), jnp.float32)
        return 0
    lax.fori_loop(0, 1000, _z, 0)

    for k in range(5):
        pltpu.sync_copy(zb, acc_sh.at[pl.ds(s * 625 + k * 125, 125)])
    plsc.subcore_barrier()

    pltpu.sync_copy(src_hbm.at[pl.ds(wid * CPW, CPW)], sidx)
    pltpu.sync_copy(dst_hbm.at[pl.ds(wid * CPW, CPW)], didx)

    def _chunk(g, _):
        pltpu.sync_copy(y_hbm.at[sidx.at[g]], rows)
        pltpu.sync_copy(rows, acc_sh.at[didx.at[g]], add=True)
        return 0
    lax.fori_loop(0, CPW, _chunk, 0)
    plsc.subcore_barrier()

    for k in range(5):
        off = s * 625 + k * 125
        pltpu.sync_copy(acc_sh.at[pl.ds(off, 125)], zb)
        pltpu.sync_copy(zb, acc_out.at[c, pl.ds(off, 125)])


def _combine_body(acc_ref, y_ref, parts_ref, b_ref, out_ref):
    deg = 1.0 + parts_ref[0] + parts_ref[1]
    dis = lax.rsqrt(deg)
    out_ref[...] = dis * (acc_ref[0] + acc_ref[1] + y_ref[...]) + b_ref[...]


_combine = pl.pallas_call(
    _combine_body,
    out_shape=jax.ShapeDtypeStruct((N, H), jnp.float32),
)


def kernel(x, edge_index, W, b):
    ei = edge_index.astype(jnp.int32)
    src2d = ei[0].reshape(E // CHUNK, CHUNK)
    dst2d = ei[1].reshape(E // CHUNK, CHUNK)

    parts = _deg_kernel(dst2d)
    parts3 = parts.reshape(NC, N, 1)
    y = _dense(x, W, parts3)
    accs = _msg_kernel(y, src2d, dst2d)
    return _combine(accs, y, parts3, b.reshape(1, H))


# speedup vs baseline: 33.0493x; 1.2170x over previous
import functools

import jax
import jax.numpy as jnp
from jax import lax
from jax.experimental import pallas as pl
from jax.experimental.pallas import tpu as pltpu
from jax.experimental.pallas import tpu_sc as plsc

N = 10000
E = 320000
F = 128
H = 128

NC = 2
NS = 16
NW = NC * NS
CHUNK = 80
EPW = E // NW
CPW = EPW // CHUNK

_MESH = plsc.VectorSubcoreMesh(core_axis_name="c", subcore_axis_name="s")
_SC_PARAMS = pltpu.CompilerParams(use_tc_tiling_on_sc=False)


@functools.partial(
    pl.kernel,
    out_type=jax.ShapeDtypeStruct((NC, N), jnp.float32),
    mesh=_MESH,
    compiler_params=_SC_PARAMS,
    scratch_types=[
        pltpu.VMEM((CPW, CHUNK), jnp.int32),
        pltpu.VMEM((CHUNK,), jnp.float32),
        pltpu.VMEM((1024,), jnp.float32),
        pltpu.VMEM_SHARED((N,), jnp.float32),
        pltpu.SemaphoreType.DMA,
    ],
)
def _deg_kernel(dst_hbm, deg_out, idx_v, ones_v, zb_v, deg_sh, sem):
    c = lax.axis_index("c")
    s = lax.axis_index("s")
    wid = s * NC + c

    def _z(i, _):
        zb_v[pl.ds(i * 16, 16)] = jnp.zeros((16,), jnp.float32)
        return 0
    lax.fori_loop(0, 64, _z, 0)
    for i in range(CHUNK // 16):
        ones_v[pl.ds(i * 16, 16)] = jnp.ones((16,), jnp.float32)

    @pl.when(s < 10)
    def _():
        pltpu.sync_copy(zb_v.at[pl.ds(0, 1000)], deg_sh.at[pl.ds(s * 1000, 1000)])
    plsc.subcore_barrier()

    pltpu.sync_copy(dst_hbm.at[pl.ds(wid * CPW, CPW)], idx_v)

    def _chunk(g, _):
        pltpu.sync_copy(ones_v, deg_sh.at[idx_v.at[g]], add=True)
        return 0
    lax.fori_loop(0, CPW, _chunk, 0)
    plsc.subcore_barrier()

    @pl.when(s < 10)
    def _():
        pltpu.sync_copy(deg_sh.at[pl.ds(s * 1000, 1000)], zb_v.at[pl.ds(0, 1000)])
        pltpu.sync_copy(zb_v.at[pl.ds(0, 1000)], deg_out.at[c, pl.ds(s * 1000, 1000)])


def _dense_body(x_ref, w_ref, parts_ref, y_ref):
    deg = 1.0 + parts_ref[0] + parts_ref[1]
    dis = lax.rsqrt(deg)
    xw = jnp.dot(x_ref[...], w_ref[...], preferred_element_type=jnp.float32)
    y_ref[...] = xw * dis


_dense = pl.pallas_call(
    _dense_body,
    out_shape=jax.ShapeDtypeStruct((N, H), jnp.float32),
)


@functools.partial(
    pl.kernel,
    out_type=jax.ShapeDtypeStruct((NC, N, H), jnp.float32),
    mesh=_MESH,
    compiler_params=_SC_PARAMS,
    scratch_types=[
        pltpu.VMEM((CPW, CHUNK), jnp.int32),
        pltpu.VMEM((CPW, CHUNK), jnp.int32),
        pltpu.VMEM((2, CHUNK, H), jnp.float32),
        pltpu.VMEM_SHARED((N, H), jnp.float32),
        pltpu.SemaphoreType.DMA,
        pltpu.SemaphoreType.DMA,
    ],
)
def _msg_kernel(y_hbm, src_hbm, dst_hbm, acc_out, sidx, didx, rows, acc_sh,
                sem0, sem1):
    c = lax.axis_index("c")
    s = lax.axis_index("s")
    wid = s * NC + c

    def _z(t, _):
        i = t // 8
        j = t % 8
        rows[0, i, pl.ds(j * 16, 16)] = jnp.zeros((16,), jnp.float32)
        return 0
    lax.fori_loop(0, CHUNK * 8, _z, 0)

    for k in range(8):
        blk = s * 8 + k

        @pl.when(blk < 125)
        def _():
            pltpu.sync_copy(rows.at[0], acc_sh.at[pl.ds(blk * CHUNK, CHUNK)])
    plsc.subcore_barrier()

    pltpu.sync_copy(src_hbm.at[pl.ds(wid * CPW, CPW)], sidx)
    pltpu.sync_copy(dst_hbm.at[pl.ds(wid * CPW, CPW)], didx)

    pltpu.sync_copy(y_hbm.at[sidx.at[0]], rows.at[0])

    def _pair(i, _):
        g = 2 * i
        d1 = pltpu.async_copy(y_hbm.at[sidx.at[g + 1]], rows.at[1], sem1)
        pltpu.sync_copy(rows.at[0], acc_sh.at[didx.at[g]], add=True)
        d1.wait()
        d2 = pltpu.async_copy(y_hbm.at[sidx.at[g + 2]], rows.at[0], sem0)
        pltpu.sync_copy(rows.at[1], acc_sh.at[didx.at[g + 1]], add=True)
        d2.wait()
        return 0
    lax.fori_loop(0, (CPW - 1) // 2, _pair, 0)

    pltpu.sync_copy(rows.at[0], acc_sh.at[didx.at[CPW - 1]], add=True)
    plsc.subcore_barrier()

    for k in range(8):
        blk = s * 8 + k

        @pl.when(blk < 125)
        def _():
            pltpu.sync_copy(acc_sh.at[pl.ds(blk * CHUNK, CHUNK)], rows.at[0])
            pltpu.sync_copy(rows.at[0], acc_out.at[c, pl.ds(blk * CHUNK, CHUNK)])


def _combine_body(acc_ref, y_ref, parts_ref, b_ref, out_ref):
    deg = 1.0 + parts_ref[0] + parts_ref[1]
    dis = lax.rsqrt(deg)
    out_ref[...] = dis * (acc_ref[0] + acc_ref[1] + y_ref[...]) + b_ref[...]


_combine = pl.pallas_call(
    _combine_body,
    out_shape=jax.ShapeDtypeStruct((N, H), jnp.float32),
)


def kernel(x, edge_index, W, b):
    ei = edge_index.astype(jnp.int32)
    src2d = ei[0].reshape(E // CHUNK, CHUNK)
    dst2d = ei[1].reshape(E // CHUNK, CHUNK)

    parts = _deg_kernel(dst2d)
    parts3 = parts.reshape(NC, N, 1)
    y = _dense(x, W, parts3)
    accs = _msg_kernel(y, src2d, dst2d)
    return _combine(accs, y, parts3, b.reshape(1, H))
